# baseline (speedup 1.0000x reference)
import functools

import jax
import jax.numpy as jnp
import numpy as np
from jax import lax
from jax.experimental import pallas as pl
from jax.experimental.pallas import tpu as pltpu
from jax.experimental.pallas import tpu_sc as plsc

_H = 384
_W = 384
_B = 4
_Z = 64
_BANDS = 8
_BH = _H // _BANDS
_CELL = 48
_TBL = 96
_NC = 2
_NS = 16
_L = 16


def _sc_paint_pool(zinfo, t0, t1, num_cores):
    mesh = plsc.VectorSubcoreMesh(
        core_axis_name="c", subcore_axis_name="s",
        num_cores=num_cores, num_subcores=_NS)

    @functools.partial(
        pl.kernel,
        out_type=jax.ShapeDtypeStruct((num_cores * _NS, _L), jnp.float32),
        mesh=mesh,
        scratch_types=[
            pltpu.VMEM((4, _Z), jnp.int32),
            pltpu.VMEM((_TBL,), jnp.float32),
            pltpu.VMEM((_TBL,), jnp.float32),
            pltpu.VMEM((_BH * _W,), jnp.int32),
            pltpu.VMEM((_L,), jnp.float32),
        ],
        compiler_params=pltpu.CompilerParams(
            use_tc_tiling_on_sc=False, needs_layout_passes=False),
    )
    def body(zinfo_hbm, t0_hbm, t1_hbm, out_hbm, zv, t0v, t1v, winv, outv):
        wid = lax.axis_index("s") * num_cores + lax.axis_index("c")
        b = wid // _BANDS
        band = wid % _BANDS
        y0 = band * _BH

        pltpu.sync_copy(zinfo_hbm.at[b], zv)
        pltpu.sync_copy(t0_hbm.at[b], t0v)
        pltpu.sync_copy(t1_hbm.at[b], t1v)

        lanes = lax.iota(jnp.int32, _L)
        sent = jnp.full((_L,), _Z, jnp.int32)

        def init_row(y, carry):
            for c in range(_W // _L):
                winv[pl.ds(y * _W + c * _L, _L)] = sent
            return carry
        lax.fori_loop(0, _BH, init_row, 0)

        def paint_zone(z, sx, ex, ylo, yhi):
            @pl.when((ylo < yhi) & (sx < ex))
            def _():
                sxs = jnp.full((_L,), sx, jnp.int32)
                exs = jnp.full((_L,), ex, jnp.int32)
                zs = jnp.full((_L,), z, jnp.int32)
                cxlo = sx // _L
                cxhi = (ex + _L - 1) // _L

                def ybody(y, c2):
                    yb = jnp.full((_L,), y * _W, jnp.int32)

                    def cbody(cx, c3):
                        xs = cx * _L + lanes
                        m = (xs >= sxs) & (xs < exs)
                        plsc.store_scatter(winv, [yb + xs], zs, mask=m)
                        return c3
                    lax.fori_loop(cxlo, cxhi, cbody, 0)
                    return c2
                lax.fori_loop(ylo, yhi, ybody, 0)

        for chunk in range(_Z // _L):
            syv = zv[0, pl.ds(chunk * _L, _L)]
            sxv = zv[1, pl.ds(chunk * _L, _L)]
            eyv = zv[2, pl.ds(chunk * _L, _L)]
            exv = zv[3, pl.ds(chunk * _L, _L)]
            ylov = jnp.maximum(syv - y0, 0)
            yhiv = jnp.minimum(eyv - y0, _BH)
            for lane in range(_L):
                paint_zone(chunk * _L + lane, sxv[lane], exv[lane],
                           ylov[lane], yhiv[lane])

        zerov = jnp.zeros((_L,), jnp.float32)

        def pool_row(y, accs):
            out = []
            for j in range(_BANDS):
                a0 = accs[2 * j]
                a1 = accs[2 * j + 1]
                for c in range(_CELL // _L):
                    w = winv[pl.ds(y * _W + j * _CELL + c * _L, _L)]
                    a0 = a0 + plsc.load_gather(t0v, [w])
                    a1 = a1 + plsc.load_gather(t1v, [w])
                out.append(a0)
                out.append(a1)
            return tuple(out)
        accs = lax.fori_loop(0, _BH, pool_row, (zerov,) * (2 * _BANDS))

        ov = zerov
        for j in range(_BANDS):
            ov = jnp.where(lanes == j, jnp.sum(accs[2 * j]), ov)
            ov = jnp.where(lanes == (_BANDS + j), jnp.sum(accs[2 * j + 1]), ov)
        outv[...] = ov
        pltpu.sync_copy(outv, out_hbm.at[wid])

    return body(zinfo, t0, t1)


def _conv_body(sums_ref, w1_ref, b1_ref, w2_ref, b2_ref, s_ref, out_ref):
    s = sums_ref[...]
    x = jnp.log1p(jnp.maximum(s, 0.0) * (1.0 / (_CELL * _CELL)))
    w1 = w1_ref[...]
    b1 = b1_ref[...]
    b2 = b2_ref[...]
    for b in range(_B):
        xb = x[:, b * 64:(b + 1) * 64]
        h = jnp.dot(w1, xb, preferred_element_type=jnp.float32) + b1
        h = h * jax.nn.sigmoid(h)
        acc = jnp.broadcast_to(b2, (32, 64))
        for k in range(9):
            g = jnp.dot(w2_ref[k], h, preferred_element_type=jnp.float32)
            acc = acc + jnp.dot(g, s_ref[k], preferred_element_type=jnp.float32)
        out_ref[b] = acc


def _shift_mats():
    s = np.zeros((9, 64, 64), np.float32)
    for dy in range(3):
        for dx in range(3):
            k = dy * 3 + dx
            for y in range(8):
                for x in range(8):
                    yy, xx = y + dy - 1, x + dx - 1
                    if 0 <= yy < 8 and 0 <= xx < 8:
                        s[k, yy * 8 + xx, y * 8 + x] = 1.0
    return s


_S = _shift_mats()


def kernel(hist_BZ2, mask_BZ, fr_BZ4, H, W, W1, b1, W2, b2):
    fr = fr_BZ4.astype(jnp.int32)
    sy = jnp.maximum(fr[..., 0], 0)
    sx = jnp.maximum(fr[..., 1], 0)
    ey = jnp.minimum(fr[..., 2], H)
    ex = jnp.minimum(fr[..., 3], W)
    valid = (ey > sy) & (ex > sx) & (mask_BZ > 0)
    zero = jnp.zeros((), jnp.int32)
    zinfo = jnp.stack([
        jnp.where(valid, sy, zero),
        jnp.where(valid, sx, zero),
        jnp.where(valid, ey, zero),
        jnp.where(valid, ex, zero),
    ], axis=1)

    pad = jnp.zeros((_B, _TBL - _Z), jnp.float32)
    t0 = jnp.concatenate([hist_BZ2[..., 0], pad], axis=1)
    t1 = jnp.concatenate([hist_BZ2[..., 1], pad], axis=1)

    sums_a = _sc_paint_pool(zinfo[:2], t0[:2], t1[:2], 1)
    sums_b = _sc_paint_pool(zinfo[2:], t0[2:], t1[2:], 1)
    sums = jnp.concatenate([sums_a, sums_b], axis=0)
    s3 = sums.reshape(_B, _BANDS, _L)
    ch0 = s3[:, :, :_BANDS].reshape(_B, 64)
    ch1 = s3[:, :, _BANDS:].reshape(_B, 64)
    sums2 = jnp.stack([ch0, ch1]).reshape(2, _B * 64)

    w2m = jnp.transpose(W2, (2, 3, 0, 1)).reshape(9, 32, 64)
    out = pl.pallas_call(
        _conv_body,
        out_shape=jax.ShapeDtypeStruct((_B, 32, 64), jnp.float32),
    )(sums2, W1.reshape(64, 2), b1.reshape(64, 1), w2m,
      b2.reshape(32, 1), jnp.asarray(_S))
    return out.reshape(_B, 32, 8, 8)


# speedup vs baseline: 1.5389x; 1.5389x over previous
import functools

import jax
import jax.numpy as jnp
import numpy as np
from jax import lax
from jax.experimental import pallas as pl
from jax.experimental.pallas import tpu as pltpu
from jax.experimental.pallas import tpu_sc as plsc

_H = 384
_W = 384
_B = 4
_Z = 64
_BANDS = 8
_BH = _H // _BANDS
_CELL = 48
_TBL = 96
_NC = 2
_NS = 16
_L = 16


def _sc_paint_pool(zinfo, t0, t1, num_cores):
    mesh = plsc.VectorSubcoreMesh(
        core_axis_name="c", subcore_axis_name="s",
        num_cores=num_cores, num_subcores=_NS)

    @functools.partial(
        pl.kernel,
        out_type=jax.ShapeDtypeStruct((num_cores * _NS, _L), jnp.float32),
        mesh=mesh,
        scratch_types=[
            pltpu.VMEM((4, _Z), jnp.int32),
            pltpu.VMEM((_TBL,), jnp.float32),
            pltpu.VMEM((_TBL,), jnp.float32),
            pltpu.VMEM((_BH * _W,), jnp.int32),
            pltpu.VMEM((_L,), jnp.float32),
        ],
        compiler_params=pltpu.CompilerParams(
            use_tc_tiling_on_sc=False, needs_layout_passes=False),
    )
    def body(zinfo_hbm, t0_hbm, t1_hbm, out_hbm, zv, t0v, t1v, winv, outv):
        wid = lax.axis_index("s") * num_cores + lax.axis_index("c")
        b = wid // _BANDS
        band = wid % _BANDS
        y0 = band * _BH

        pltpu.sync_copy(zinfo_hbm.at[b], zv)
        pltpu.sync_copy(t0_hbm.at[b], t0v)
        pltpu.sync_copy(t1_hbm.at[b], t1v)

        lanes = lax.iota(jnp.int32, _L)
        sent = jnp.full((_L,), _Z, jnp.int32)

        with jax.named_scope("sc_init"):
            def init_row(y, carry):
                for c in range(_W // _L):
                    winv[pl.ds(y * _W + c * _L, _L)] = sent
                return carry
            lax.fori_loop(0, _BH, init_row, 0)

        def paint_zone(z, sx, ex, ylo, yhi):
            @pl.when((ylo < yhi) & (sx < ex))
            def _():
                sxs = jnp.full((_L,), sx, jnp.int32)
                exs = jnp.full((_L,), ex, jnp.int32)
                zs = jnp.full((_L,), z, jnp.int32)
                cxlo = sx // _L
                cxhi = (ex + _L - 1) // _L

                def ybody(y, c2):
                    yb = jnp.full((_L,), y * _W, jnp.int32)

                    def cbody(cx, c3):
                        xs = cx * _L + lanes
                        m = (xs >= sxs) & (xs < exs)
                        plsc.store_scatter(winv, [yb + xs], zs, mask=m)
                        return c3
                    lax.fori_loop(cxlo, cxhi, cbody, 0)
                    return c2
                lax.fori_loop(ylo, yhi, ybody, 0)

        with jax.named_scope("sc_paint"):
            for chunk in range(_Z // _L):
                syv = zv[0, pl.ds(chunk * _L, _L)]
                sxv = zv[1, pl.ds(chunk * _L, _L)]
                eyv = zv[2, pl.ds(chunk * _L, _L)]
                exv = zv[3, pl.ds(chunk * _L, _L)]
                ylov = jnp.maximum(syv - y0, 0)
                yhiv = jnp.minimum(eyv - y0, _BH)
                for lane in range(_L):
                    paint_zone(chunk * _L + lane, sxv[lane], exv[lane],
                               ylov[lane], yhiv[lane])

        zerov = jnp.zeros((_L,), jnp.float32)

        def pool_row(y, accs):
            out = []
            for j in range(_BANDS):
                a0 = accs[2 * j]
                a1 = accs[2 * j + 1]
                for c in range(_CELL // _L):
                    w = winv[pl.ds(y * _W + j * _CELL + c * _L, _L)]
                    a0 = a0 + plsc.load_gather(t0v, [w])
                    a1 = a1 + plsc.load_gather(t1v, [w])
                out.append(a0)
                out.append(a1)
            return tuple(out)
        with jax.named_scope("sc_pool"):
            accs = lax.fori_loop(0, _BH, pool_row, (zerov,) * (2 * _BANDS))

        ov = zerov
        for j in range(_BANDS):
            ov = jnp.where(lanes == j, jnp.sum(accs[2 * j]), ov)
            ov = jnp.where(lanes == (_BANDS + j), jnp.sum(accs[2 * j + 1]), ov)
        outv[...] = ov
        pltpu.sync_copy(outv, out_hbm.at[wid])

    return body(zinfo, t0, t1)


def _conv_body(sums_ref, w1_ref, b1_ref, w2_ref, b2_ref, s_ref, out_ref):
    s = sums_ref[...]
    x = jnp.log1p(jnp.maximum(s, 0.0) * (1.0 / (_CELL * _CELL)))
    w1 = w1_ref[...]
    b1 = b1_ref[...]
    b2 = b2_ref[...]
    for b in range(_B):
        xb = x[:, b * 64:(b + 1) * 64]
        h = jnp.dot(w1, xb, preferred_element_type=jnp.float32) + b1
        h = h * jax.nn.sigmoid(h)
        acc = jnp.broadcast_to(b2, (32, 64))
        for k in range(9):
            g = jnp.dot(w2_ref[k], h, preferred_element_type=jnp.float32)
            acc = acc + jnp.dot(g, s_ref[k], preferred_element_type=jnp.float32)
        out_ref[b] = acc


def _shift_mats():
    s = np.zeros((9, 64, 64), np.float32)
    for dy in range(3):
        for dx in range(3):
            k = dy * 3 + dx
            for y in range(8):
                for x in range(8):
                    yy, xx = y + dy - 1, x + dx - 1
                    if 0 <= yy < 8 and 0 <= xx < 8:
                        s[k, yy * 8 + xx, y * 8 + x] = 1.0
    return s


_S = _shift_mats()


def kernel(hist_BZ2, mask_BZ, fr_BZ4, H, W, W1, b1, W2, b2):
    fr = fr_BZ4.astype(jnp.int32)
    sy = jnp.maximum(fr[..., 0], 0)
    sx = jnp.maximum(fr[..., 1], 0)
    ey = jnp.minimum(fr[..., 2], H)
    ex = jnp.minimum(fr[..., 3], W)
    valid = (ey > sy) & (ex > sx) & (mask_BZ > 0)
    zero = jnp.zeros((), jnp.int32)
    zinfo = jnp.stack([
        jnp.where(valid, sy, zero),
        jnp.where(valid, sx, zero),
        jnp.where(valid, ey, zero),
        jnp.where(valid, ex, zero),
    ], axis=1)

    pad = jnp.zeros((_B, _TBL - _Z), jnp.float32)
    t0 = jnp.concatenate([hist_BZ2[..., 0], pad], axis=1)
    t1 = jnp.concatenate([hist_BZ2[..., 1], pad], axis=1)

    sums = _sc_paint_pool(zinfo, t0, t1, _NC)
    s3 = sums.reshape(_B, _BANDS, _L)
    ch0 = s3[:, :, :_BANDS].reshape(_B, 64)
    ch1 = s3[:, :, _BANDS:].reshape(_B, 64)
    sums2 = jnp.stack([ch0, ch1]).reshape(2, _B * 64)

    w2m = jnp.transpose(W2, (2, 3, 0, 1)).reshape(9, 32, 64)
    out = pl.pallas_call(
        _conv_body,
        out_shape=jax.ShapeDtypeStruct((_B, 32, 64), jnp.float32),
    )(sums2, W1.reshape(64, 2), b1.reshape(64, 1), w2m,
      b2.reshape(32, 1), jnp.asarray(_S))
    return out.reshape(_B, 32, 8, 8)


# speedup vs baseline: 1.7979x; 1.1683x over previous
import functools

import jax
import jax.numpy as jnp
import numpy as np
from jax import lax
from jax.experimental import pallas as pl
from jax.experimental.pallas import tpu as pltpu
from jax.experimental.pallas import tpu_sc as plsc

_H = 384
_W = 384
_B = 4
_Z = 64
_BANDS = 8
_BH = _H // _BANDS
_CELL = 48
_TBL = 96
_ZP = 528
_NC = 2
_NS = 16
_L = 16


def _sc_paint_pool(zpack, num_cores):
    mesh = plsc.VectorSubcoreMesh(
        core_axis_name="c", subcore_axis_name="s",
        num_cores=num_cores, num_subcores=_NS)

    @functools.partial(
        pl.kernel,
        out_type=jax.ShapeDtypeStruct((num_cores * _NS, _L), jnp.float32),
        mesh=mesh,
        scratch_types=[
            pltpu.VMEM((_ZP + 2 * _TBL,), jnp.int32),
            pltpu.VMEM((_BH * _W,), jnp.int32),
            pltpu.VMEM((_L,), jnp.float32),
        ],
        compiler_params=pltpu.CompilerParams(
            use_tc_tiling_on_sc=False, needs_layout_passes=False),
    )
    def body(zpack_hbm, out_hbm, zv, winv, outv):
        wid = lax.axis_index("s") * num_cores + lax.axis_index("c")
        b = wid // _BANDS
        band = wid % _BANDS
        y0 = band * _BH

        pltpu.sync_copy(zpack_hbm.at[b], zv)

        lanes = lax.iota(jnp.int32, _L)
        sent = jnp.full((_L,), _Z, jnp.int32)

        with jax.named_scope("sc_init"):
            def init_row(y, carry):
                for c in range(_W // _L):
                    winv[pl.ds(y * _W + c * _L, _L)] = sent
                return carry
            lax.fori_loop(0, _BH, init_row, 0)

        with jax.named_scope("sc_paint"):
            def zbody(z, carry):
                f = zv[pl.ds(z * 8, _L)]
                sy = f[0]
                sx = f[1]
                ey = f[2]
                ex = f[3]
                ylo = jnp.maximum(sy - y0, 0)
                yhi = jnp.minimum(ey - y0, _BH)

                @pl.when((ylo < yhi) & (sx < ex))
                def _():
                    sxs = jnp.full((_L,), sx, jnp.int32)
                    exs = jnp.full((_L,), ex, jnp.int32)
                    zs = jnp.full((_L,), z, jnp.int32)
                    cxlo = sx // _L
                    cxhi = (ex + _L - 1) // _L

                    def ybody(y, c2):
                        yb = jnp.full((_L,), y * _W, jnp.int32)

                        def cbody(cx, c3):
                            xs = cx * _L + lanes
                            m = (xs >= sxs) & (xs < exs)
                            plsc.store_scatter(winv, [yb + xs], zs, mask=m)
                            return c3
                        lax.fori_loop(cxlo, cxhi, cbody, 0)
                        return c2
                    lax.fori_loop(ylo, yhi, ybody, 0)
                return carry
            lax.fori_loop(0, _Z, zbody, 0)

        zerov = jnp.zeros((_L,), jnp.float32)

        def pool_row(y, accs):
            out = []
            for j in range(_BANDS):
                a0 = accs[2 * j]
                a1 = accs[2 * j + 1]
                for c in range(_CELL // _L):
                    w = winv[pl.ds(y * _W + j * _CELL + c * _L, _L)]
                    a0 = a0 + plsc.bitcast(
                        plsc.load_gather(zv, [w + _ZP]), jnp.float32)
                    a1 = a1 + plsc.bitcast(
                        plsc.load_gather(zv, [w + (_ZP + _TBL)]), jnp.float32)
                out.append(a0)
                out.append(a1)
            return tuple(out)
        with jax.named_scope("sc_pool"):
            accs = lax.fori_loop(0, _BH, pool_row, (zerov,) * (2 * _BANDS))

        ov = zerov
        for j in range(_BANDS):
            ov = jnp.where(lanes == j, jnp.sum(accs[2 * j]), ov)
            ov = jnp.where(lanes == (_BANDS + j), jnp.sum(accs[2 * j + 1]), ov)
        outv[...] = ov
        pltpu.sync_copy(outv, out_hbm.at[wid])

    return body(zpack)


def _conv_body(sums_ref, w1_ref, b1_ref, w2_ref, b2_ref, s_ref, out_ref):
    s = sums_ref[...]
    x = jnp.log1p(jnp.maximum(s, 0.0) * (1.0 / (_CELL * _CELL)))
    w1 = w1_ref[...]
    b1 = b1_ref[...]
    b2 = b2_ref[...]
    for b in range(_B):
        xb = x[:, b * 64:(b + 1) * 64]
        h = jnp.dot(w1, xb, preferred_element_type=jnp.float32) + b1
        h = h * jax.nn.sigmoid(h)
        acc = jnp.broadcast_to(b2, (32, 64))
        for k in range(9):
            g = jnp.dot(w2_ref[k], h, preferred_element_type=jnp.float32)
            acc = acc + jnp.dot(g, s_ref[k], preferred_element_type=jnp.float32)
        out_ref[b] = acc


def _shift_mats():
    s = np.zeros((9, 64, 64), np.float32)
    for dy in range(3):
        for dx in range(3):
            k = dy * 3 + dx
            for y in range(8):
                for x in range(8):
                    yy, xx = y + dy - 1, x + dx - 1
                    if 0 <= yy < 8 and 0 <= xx < 8:
                        s[k, yy * 8 + xx, y * 8 + x] = 1.0
    return s


_S = _shift_mats()


def kernel(hist_BZ2, mask_BZ, fr_BZ4, H, W, W1, b1, W2, b2):
    fr = fr_BZ4.astype(jnp.int32)
    sy = jnp.maximum(fr[..., 0], 0)
    sx = jnp.maximum(fr[..., 1], 0)
    ey = jnp.minimum(fr[..., 2], H)
    ex = jnp.minimum(fr[..., 3], W)
    valid = (ey > sy) & (ex > sx) & (mask_BZ > 0)
    zero = jnp.zeros((), jnp.int32)
    zinfo = jnp.stack([
        jnp.where(valid, sy, zero),
        jnp.where(valid, sx, zero),
        jnp.where(valid, ey, zero),
        jnp.where(valid, ex, zero),
    ], axis=-1)
    zrec = jnp.concatenate(
        [zinfo, jnp.zeros((_B, _Z, 4), jnp.int32)], axis=-1).reshape(_B, _Z * 8)
    zrec = jnp.concatenate(
        [zrec, jnp.zeros((_B, _ZP - _Z * 8), jnp.int32)], axis=1)

    pad = jnp.zeros((_B, _TBL - _Z), jnp.float32)
    t0 = jnp.concatenate([hist_BZ2[..., 0], pad], axis=1)
    t1 = jnp.concatenate([hist_BZ2[..., 1], pad], axis=1)
    zpack = jnp.concatenate([
        zrec,
        lax.bitcast_convert_type(t0, jnp.int32),
        lax.bitcast_convert_type(t1, jnp.int32),
    ], axis=1)

    sums = _sc_paint_pool(zpack, _NC)
    s3 = sums.reshape(_B, _BANDS, _L)
    ch0 = s3[:, :, :_BANDS].reshape(_B, 64)
    ch1 = s3[:, :, _BANDS:].reshape(_B, 64)
    sums2 = jnp.stack([ch0, ch1]).reshape(2, _B * 64)

    w2m = jnp.transpose(W2, (2, 3, 0, 1)).reshape(9, 32, 64)
    out = pl.pallas_call(
        _conv_body,
        out_shape=jax.ShapeDtypeStruct((_B, 32, 64), jnp.float32),
    )(sums2, W1.reshape(64, 2), b1.reshape(64, 1), w2m,
      b2.reshape(32, 1), jnp.asarray(_S))
    return out.reshape(_B, 32, 8, 8)
